# Initial kernel scaffold; baseline (speedup 1.0000x reference)
import jax
import jax.numpy as jnp
from jax.experimental import pallas as pl


def kernel(x, neighbor, W1, b1, W2, b2, Wc, bc, gamma, beta):
    raise NotImplementedError("write your pallas kernel here")


# speedup vs baseline: 1.5464x; 1.5464x over previous
import functools

import jax
import jax.numpy as jnp
from jax.experimental import pallas as pl
from jax.experimental.pallas import tpu as pltpu

ALPHA = 0.1
Q = 1.0 - ALPHA
EPS = 1e-5
BLK = 400


def _bn_relu_global(t, mu, rs, gamma, beta):
    return jnp.maximum(gamma * (t - mu) * rs + beta, 0.0)


def _global_stats(part, n_elems):
    s = jnp.sum(part[:, 0, 0])
    ss = jnp.sum(part[:, 0, 1])
    mu = s / n_elems
    var = jnp.maximum(ss / n_elems - mu * mu, 0.0)
    return mu, jax.lax.rsqrt(var + EPS)


def _partial_vec(t):
    s = jnp.sum(t)
    ss = jnp.sum(t * t)
    lane = jax.lax.broadcasted_iota(jnp.int32, (1, 1, 128), 2)
    return jnp.where(lane == 0, s, jnp.where(lane == 1, ss, 0.0))


def _pernode_bn_relu(t, gamma, beta):
    mu = jnp.mean(t, axis=(1, 2), keepdims=True)
    var = jnp.maximum(jnp.mean(t * t, axis=(1, 2), keepdims=True) - mu * mu, 0.0)
    rs = jax.lax.rsqrt(var + EPS)
    return jnp.maximum(gamma * (t - mu) * rs + beta, 0.0)


def _pass1_body(x_ref, nb_ref, w1_ref, b1_ref,
                naggp_ref, h_ref, x1p_ref, part_ref):
    B, K, D = nb_ref.shape
    xb = x_ref[...]
    nb = nb_ref[...]
    w1 = w1_ref[...]
    b1 = b1_ref[...]
    h = jnp.dot(xb, w1, preferred_element_type=jnp.float32) + b1
    nh = jnp.dot(nb.reshape(B * K, D), w1,
                 preferred_element_type=jnp.float32) + b1
    nh3 = nh.reshape(B, K, h.shape[-1])
    x1p = h + Q * jnp.sum(nh3, axis=1)
    naggp_ref[...] = nh3 + Q * h[:, None, :]
    h_ref[...] = h
    x1p_ref[...] = x1p
    part_ref[...] = _partial_vec(x1p)


def _pass2_body(naggp_ref, h_ref, x1p_ref, part_ref, w2_ref, b2_ref, gb_ref,
                nagg2p_ref, h2_ref, x3p_ref, part2_ref, *, n_elems):
    B, K, H1 = naggp_ref.shape
    gamma = gb_ref[0, 0]
    beta = gb_ref[0, 1]
    mu, rs = _global_stats(part_ref[...], n_elems)
    x1 = _bn_relu_global(x1p_ref[...], mu, rs, gamma, beta)
    naggp = naggp_ref[...]
    nagg = _pernode_bn_relu(naggp, gamma, beta)
    h = h_ref[...]
    x2 = Q * (x1 + jnp.sum(nagg, axis=1)) + ALPHA * h
    nh = naggp - Q * h[:, None, :]
    n2 = Q * (nagg + x1[:, None, :]) + ALPHA * nh
    w2 = w2_ref[...]
    b2 = b2_ref[...]
    h2 = jnp.dot(x2, w2, preferred_element_type=jnp.float32) + b2
    nh2 = jnp.dot(n2.reshape(B * K, H1), w2,
                  preferred_element_type=jnp.float32) + b2
    nh23 = nh2.reshape(B, K, h2.shape[-1])
    x3p = h2 + Q * jnp.sum(nh23, axis=1)
    nagg2p_ref[...] = nh23 + Q * h2[:, None, :]
    h2_ref[...] = h2
    x3p_ref[...] = x3p
    part2_ref[...] = _partial_vec(x3p)


def _pass3_body(nagg2p_ref, h2_ref, x3p_ref, part_ref, wc_ref, bc_ref, gb_ref,
                out_ref, *, n_elems):
    gamma = gb_ref[0, 0]
    beta = gb_ref[0, 1]
    mu, rs = _global_stats(part_ref[...], n_elems)
    x3 = _bn_relu_global(x3p_ref[...], mu, rs, gamma, beta)
    nagg2 = _pernode_bn_relu(nagg2p_ref[...], gamma, beta)
    h2 = h2_ref[...]
    x4 = Q * (x3 + jnp.sum(nagg2, axis=1)) + ALPHA * h2
    x4 = jnp.where(jnp.isnan(x4), 0.0, x4)
    out_ref[...] = jnp.dot(x4, wc_ref[...],
                           preferred_element_type=jnp.float32) + bc_ref[...]


def kernel(x, neighbor, W1, b1, W2, b2, Wc, bc, gamma, beta):
    N, _, D = x.shape
    K = neighbor.shape[1]
    H1 = W1.shape[1]
    H2 = W2.shape[1]
    C = Wc.shape[1]
    B = BLK
    G = N // B
    f32 = jnp.float32

    x2d = x.reshape(N, D)
    nb3 = neighbor.reshape(N, K, D)
    b1r = b1.reshape(1, H1)
    b2r = b2.reshape(1, H2)
    bcr = bc.reshape(1, C)
    gb = jnp.concatenate([gamma, beta]).reshape(1, 2)

    params = pltpu.CompilerParams(dimension_semantics=("parallel",))

    naggp, h, x1p, part1 = pl.pallas_call(
        _pass1_body,
        grid=(G,),
        in_specs=[
            pl.BlockSpec((B, D), lambda i: (i, 0)),
            pl.BlockSpec((B, K, D), lambda i: (i, 0, 0)),
            pl.BlockSpec((D, H1), lambda i: (0, 0)),
            pl.BlockSpec((1, H1), lambda i: (0, 0)),
        ],
        out_specs=[
            pl.BlockSpec((B, K, H1), lambda i: (i, 0, 0)),
            pl.BlockSpec((B, H1), lambda i: (i, 0)),
            pl.BlockSpec((B, H1), lambda i: (i, 0)),
            pl.BlockSpec((1, 1, 128), lambda i: (i, 0, 0)),
        ],
        out_shape=[
            jax.ShapeDtypeStruct((N, K, H1), f32),
            jax.ShapeDtypeStruct((N, H1), f32),
            jax.ShapeDtypeStruct((N, H1), f32),
            jax.ShapeDtypeStruct((G, 1, 128), f32),
        ],
        compiler_params=params,
    )(x2d, nb3, W1, b1r)

    nagg2p, h2, x3p, part2 = pl.pallas_call(
        functools.partial(_pass2_body, n_elems=float(N * H1)),
        grid=(G,),
        in_specs=[
            pl.BlockSpec((B, K, H1), lambda i: (i, 0, 0)),
            pl.BlockSpec((B, H1), lambda i: (i, 0)),
            pl.BlockSpec((B, H1), lambda i: (i, 0)),
            pl.BlockSpec((G, 1, 128), lambda i: (0, 0, 0)),
            pl.BlockSpec((H1, H2), lambda i: (0, 0)),
            pl.BlockSpec((1, H2), lambda i: (0, 0)),
            pl.BlockSpec((1, 2), lambda i: (0, 0)),
        ],
        out_specs=[
            pl.BlockSpec((B, K, H2), lambda i: (i, 0, 0)),
            pl.BlockSpec((B, H2), lambda i: (i, 0)),
            pl.BlockSpec((B, H2), lambda i: (i, 0)),
            pl.BlockSpec((1, 1, 128), lambda i: (i, 0, 0)),
        ],
        out_shape=[
            jax.ShapeDtypeStruct((N, K, H2), f32),
            jax.ShapeDtypeStruct((N, H2), f32),
            jax.ShapeDtypeStruct((N, H2), f32),
            jax.ShapeDtypeStruct((G, 1, 128), f32),
        ],
        compiler_params=params,
    )(naggp, h, x1p, part1, W2, b2r, gb)

    out = pl.pallas_call(
        functools.partial(_pass3_body, n_elems=float(N * H2)),
        grid=(G,),
        in_specs=[
            pl.BlockSpec((B, K, H2), lambda i: (i, 0, 0)),
            pl.BlockSpec((B, H2), lambda i: (i, 0)),
            pl.BlockSpec((B, H2), lambda i: (i, 0)),
            pl.BlockSpec((G, 1, 128), lambda i: (0, 0, 0)),
            pl.BlockSpec((H2, C), lambda i: (0, 0)),
            pl.BlockSpec((1, C), lambda i: (0, 0)),
            pl.BlockSpec((1, 2), lambda i: (0, 0)),
        ],
        out_specs=pl.BlockSpec((B, C), lambda i: (i, 0)),
        out_shape=jax.ShapeDtypeStruct((N, C), f32),
        compiler_params=params,
    )(nagg2p, h2, x3p, part2, Wc, bcr, gb)

    return out
